# baseline (speedup 1.0000x reference)
import functools

import jax
import jax.numpy as jnp
from jax import lax
from jax.experimental import pallas as pl
from jax.experimental.pallas import tpu as pltpu
from jax.experimental.pallas import tpu_sc as plsc

_EPS = 1e-5

_NC = 2
_NS = 16
_NW = _NC * _NS
_CH = 128


def _sc_gather_pair(a_tbl, b_tbl, ia2d, ib2d, kk=4):
    ncr, ch = ia2d.shape
    dim = a_tbl.shape[1]
    dt = a_tbl.dtype
    e_pad = ncr * ch
    cr_per_w = ncr // _NW
    nblk = cr_per_w // kk
    mesh = plsc.VectorSubcoreMesh(core_axis_name="c", subcore_axis_name="s",
                                  num_cores=_NC, num_subcores=_NS)

    @functools.partial(
        pl.kernel,
        out_type=jax.ShapeDtypeStruct((e_pad, 2 * dim), dt),
        mesh=mesh,
        scratch_types=[
            pltpu.VMEM((kk, ch), jnp.int32),
            pltpu.VMEM((kk, ch), jnp.int32),
            pltpu.VMEM((kk * ch, dim), dt),
            pltpu.VMEM((kk * ch, dim), dt),
            pltpu.SemaphoreType.DMA,
            pltpu.SemaphoreType.DMA,
        ],
        compiler_params=pltpu.CompilerParams(use_tc_tiling_on_sc=False),
    )
    def k(a_hbm, b_hbm, ia_hbm, ib_hbm, out_hbm, ia_v, ib_v, ra_v, rb_v,
          sa, sb):
        wid = lax.axis_index("s") * _NC + lax.axis_index("c")

        def blk(b, carry):
            cr = wid * cr_per_w + b * kk
            pltpu.sync_copy(ia_hbm.at[pl.ds(cr, kk)], ia_v)
            pltpu.sync_copy(ib_hbm.at[pl.ds(cr, kk)], ib_v)
            da = [pltpu.async_copy(a_hbm.at[ia_v.at[j]],
                                   ra_v.at[pl.ds(j * ch, ch)], sa)
                  for j in range(kk)]
            db = [pltpu.async_copy(b_hbm.at[ib_v.at[j]],
                                   rb_v.at[pl.ds(j * ch, ch)], sb)
                  for j in range(kk)]
            for dsc in da:
                dsc.wait()
            for dsc in db:
                dsc.wait()
            pltpu.sync_copy(
                ra_v, out_hbm.at[pl.ds(cr * ch, kk * ch), pl.ds(0, dim)])
            pltpu.sync_copy(
                rb_v, out_hbm.at[pl.ds(cr * ch, kk * ch), pl.ds(dim, dim)])
            return carry

        lax.fori_loop(0, nblk, blk, 0)

    return k(a_tbl, b_tbl, ia2d, ib2d)


def _sc_segsum(msg, col2d, n_pad, kk=4):
    ncr, ch = col2d.shape
    dim = msg.shape[1]
    stripe = n_pad // _NS
    cr_per_core = ncr // _NC
    cr_per_w = cr_per_core // _NS
    nblk = cr_per_w // kk
    zeros = jnp.zeros((stripe, dim), jnp.float32)
    mesh = plsc.VectorSubcoreMesh(core_axis_name="c", subcore_axis_name="s",
                                  num_cores=_NC, num_subcores=_NS)

    @functools.partial(
        pl.kernel,
        out_type=jax.ShapeDtypeStruct((_NC * n_pad, dim), jnp.float32),
        mesh=mesh,
        scratch_types=[
            pltpu.VMEM((kk, ch), jnp.int32),
            pltpu.VMEM((kk * ch, dim), jnp.float32),
            pltpu.VMEM_SHARED((n_pad, dim), jnp.float32),
        ],
        compiler_params=pltpu.CompilerParams(use_tc_tiling_on_sc=False),
    )
    def k(msg_hbm, col_hbm, z_hbm, out_hbm, ci_v, mb_v, acc_sh):
        cid = lax.axis_index("c")
        sid = lax.axis_index("s")
        pltpu.sync_copy(z_hbm, acc_sh.at[pl.ds(sid * stripe, stripe)])
        plsc.subcore_barrier()

        def blk(b, carry):
            cr = cid * cr_per_core + sid * cr_per_w + b * kk
            pltpu.sync_copy(col_hbm.at[pl.ds(cr, kk)], ci_v)
            pltpu.sync_copy(msg_hbm.at[pl.ds(cr * ch, kk * ch)], mb_v)
            for j in range(kk):
                pltpu.sync_copy(mb_v.at[pl.ds(j * ch, ch)],
                                acc_sh.at[ci_v.at[j]], add=True)
            return carry

        lax.fori_loop(0, nblk, blk, 0)
        plsc.subcore_barrier()
        pltpu.sync_copy(acc_sh.at[pl.ds(sid * stripe, stripe)],
                        out_hbm.at[pl.ds(cid * n_pad + sid * stripe, stripe)])

    return k(msg, col2d, zeros)


_BN = 5000


def _stats_kernel(x_ref, sx_ref, sxx_ref):
    x = x_ref[...]

    @pl.when(pl.program_id(0) == 0)
    def _():
        sx_ref[...] = jnp.zeros_like(sx_ref)
        sxx_ref[...] = jnp.zeros_like(sxx_ref)

    sx_ref[...] += jnp.sum(x, axis=0, keepdims=True)
    sxx_ref[...] += jnp.sum(x * x, axis=0, keepdims=True)


def _node_stats(x):
    n, d = x.shape
    return pl.pallas_call(
        _stats_kernel,
        grid=(n // _BN,),
        in_specs=[pl.BlockSpec((_BN, d), lambda i: (i, 0))],
        out_specs=[pl.BlockSpec((1, d), lambda i: (0, 0)),
                   pl.BlockSpec((1, d), lambda i: (0, 0))],
        out_shape=[jax.ShapeDtypeStruct((1, d), jnp.float32),
                   jax.ShapeDtypeStruct((1, d), jnp.float32)],
    )(x)


def _node_stage_a_kernel(x_ref, sx_ref, sxx_ref, gamma_ref, beta_ref,
                         w1_ref, b1_ref, w2_ref, b2_ref, wa_ref, wb_ref,
                         cb1_ref, n_scale_ref, xout_ref, p_ref, q_ref):
    inv_n = n_scale_ref[0, 0]
    mean = sx_ref[...] * inv_n
    var = sxx_ref[...] * inv_n - mean * mean
    x = x_ref[...]
    xn = (x - mean) * lax.rsqrt(var + _EPS) * gamma_ref[...] + beta_ref[...]
    h = jnp.maximum(jnp.dot(xn, w1_ref[...], preferred_element_type=jnp.float32)
                    + b1_ref[...], 0.0)
    h = jnp.tanh(jnp.dot(h, w2_ref[...], preferred_element_type=jnp.float32)
                 + b2_ref[...])
    feat = jnp.concatenate([h, xn], axis=-1)
    xout_ref[...] = xn
    p_ref[...] = (jnp.dot(feat, wa_ref[...], preferred_element_type=jnp.float32)
                  + cb1_ref[...]).astype(jnp.bfloat16)
    q_ref[...] = jnp.dot(feat, wb_ref[...],
                         preferred_element_type=jnp.float32).astype(jnp.bfloat16)


def _node_stage_a(x, gamma, beta, w1t, b1, w2t, b2, wa, wb, cb1):
    n, d = x.shape
    hd = w1t.shape[1]
    sx, sxx = _node_stats(x)
    n_scale = jnp.full((1, 1), 1.0 / n, dtype=jnp.float32)
    small = lambda s: pl.BlockSpec(s, lambda i: (0,) * len(s))
    return pl.pallas_call(
        _node_stage_a_kernel,
        grid=(n // _BN,),
        in_specs=[
            pl.BlockSpec((_BN, d), lambda i: (i, 0)),
            small((1, d)), small((1, d)), small((1, d)), small((1, d)),
            small(w1t.shape), small(b1.shape), small(w2t.shape),
            small(b2.shape), small(wa.shape), small(wb.shape),
            small(cb1.shape), small((1, 1)),
        ],
        out_specs=[pl.BlockSpec((_BN, d), lambda i: (i, 0)),
                   pl.BlockSpec((_BN, 2 * hd), lambda i: (i, 0)),
                   pl.BlockSpec((_BN, 2 * hd), lambda i: (i, 0))],
        out_shape=[
            jax.ShapeDtypeStruct((n, d), jnp.float32),
            jax.ShapeDtypeStruct((n, 2 * hd), jnp.bfloat16),
            jax.ShapeDtypeStruct((n, 2 * hd), jnp.bfloat16),
        ],
    )(x, sx, sxx, gamma, beta, w1t, b1, w2t, b2, wa, wb, cb1, n_scale)


def _edge_mlp1_kernel(g_ref, ea_ref, we_ref, w2_ref, b2_ref, out_ref):
    g = g_ref[...].astype(jnp.float32)
    hdim = g.shape[1] // 2
    eac = lax.dot_general(ea_ref[...], we_ref[...], (((0,), (0,)), ((), ())),
                          preferred_element_type=jnp.float32)
    h = jnp.maximum(g[:, :hdim] + g[:, hdim:] + eac, 0.0)
    out_ref[...] = jnp.tanh(
        jnp.dot(h, w2_ref[...], preferred_element_type=jnp.float32) + b2_ref[...])


def _edge_mlp1(g, ea, we, w2t, b2, block_e):
    e, gdim = g.shape
    odim = w2t.shape[1]
    grid = e // block_e
    return pl.pallas_call(
        _edge_mlp1_kernel,
        grid=(grid,),
        in_specs=[
            pl.BlockSpec((block_e, gdim), lambda i: (i, 0)),
            pl.BlockSpec((ea.shape[0], block_e), lambda i: (0, i)),
            pl.BlockSpec(we.shape, lambda i: (0, 0)),
            pl.BlockSpec(w2t.shape, lambda i: (0, 0)),
            pl.BlockSpec(b2.shape, lambda i: (0, 0)),
        ],
        out_specs=pl.BlockSpec((block_e, odim), lambda i: (i, 0)),
        out_shape=jax.ShapeDtypeStruct((e, odim), jnp.float32),
    )(g, ea, we, w2t, b2)


def _edge_mlp2_kernel(g_ref, ea_ref, we_ref, w2_ref, b2_ref, out_ref):
    g = g_ref[...].astype(jnp.float32)
    hdim = g.shape[1] // 2
    eac = lax.dot_general(ea_ref[...], we_ref[...], (((0,), (0,)), ((), ())),
                          preferred_element_type=jnp.float32)
    h = jnp.maximum(g[:, :hdim] + g[:, hdim:] + eac, 0.0)
    z = lax.dot_general(w2_ref[...], h, (((1,), (1,)), ((), ())),
                        preferred_element_type=jnp.float32) + b2_ref[...]
    out_ref[...] = jax.nn.sigmoid(z)


def _edge_mlp2(g, ea, we, w2t, b2, block_e):
    e, gdim = g.shape
    grid = e // block_e
    return pl.pallas_call(
        _edge_mlp2_kernel,
        grid=(grid,),
        in_specs=[
            pl.BlockSpec((block_e, gdim), lambda i: (i, 0)),
            pl.BlockSpec((ea.shape[0], block_e), lambda i: (0, i)),
            pl.BlockSpec(we.shape, lambda i: (0, 0)),
            pl.BlockSpec(w2t.shape, lambda i: (0, 0)),
            pl.BlockSpec(b2.shape, lambda i: (0, 0)),
        ],
        out_specs=pl.BlockSpec((1, block_e), lambda i: (0, i)),
        out_shape=jax.ShapeDtypeStruct((1, e), jnp.float32),
    )(g, ea, we, w2t, b2)


def _node_stage_e_kernel(hn0_ref, hn1_ref, x_ref, wr_ref, ws_ref, eb1_ref,
                         r_ref, s_ref):
    feat2 = jnp.concatenate([hn0_ref[...] + hn1_ref[...], x_ref[...]], axis=-1)
    r_ref[...] = jnp.dot(feat2, wr_ref[...],
                         preferred_element_type=jnp.float32).astype(jnp.bfloat16)
    s_ref[...] = (jnp.dot(feat2, ws_ref[...], preferred_element_type=jnp.float32)
                  + eb1_ref[...]).astype(jnp.bfloat16)


def _node_stage_e(hn0, hn1, xn, wr, ws, eb1):
    n = xn.shape[0]
    hdim = wr.shape[1]
    small = lambda s: pl.BlockSpec(s, lambda i: (0,) * len(s))
    return pl.pallas_call(
        _node_stage_e_kernel,
        grid=(n // _BN,),
        in_specs=[
            pl.BlockSpec((_BN, hn0.shape[1]), lambda i: (i, 0)),
            pl.BlockSpec((_BN, hn1.shape[1]), lambda i: (i, 0)),
            pl.BlockSpec((_BN, xn.shape[1]), lambda i: (i, 0)),
            small(wr.shape), small(ws.shape), small(eb1.shape),
        ],
        out_specs=[pl.BlockSpec((_BN, hdim), lambda i: (i, 0)),
                   pl.BlockSpec((_BN, hdim), lambda i: (i, 0))],
        out_shape=[
            jax.ShapeDtypeStruct((n, hdim), jnp.bfloat16),
            jax.ShapeDtypeStruct((n, hdim), jnp.bfloat16),
        ],
    )(hn0, hn1, xn, wr, ws, eb1)


def kernel(x, edge_index, edge_attr, bn_gamma, bn_beta,
           in_W1, in_b1, in_W2, in_b2,
           cv_W1, cv_b1, cv_W2, cv_b2,
           eg_W1, eg_b1, eg_W2, eg_b2):
    n, d = x.shape
    e = edge_index.shape[1]
    hd = in_W1.shape[0]
    fd = hd + d

    row = edge_index[0]
    col = edge_index[1]

    wa1 = (cv_W1[:, :fd] - cv_W1[:, fd:2 * fd]).T
    wb1 = cv_W1[:, fd:2 * fd].T
    we1 = cv_W1[:, 2 * fd:].T
    wr = eg_W1[:, :fd].T
    ws = eg_W1[:, fd:2 * fd].T
    we2 = eg_W1[:, 2 * fd:].T

    xn, p, q = _node_stage_a(
        x, bn_gamma.reshape(1, d), bn_beta.reshape(1, d),
        in_W1.T, in_b1.reshape(1, hd), in_W2.T, in_b2.reshape(1, hd),
        wa1, wb1, cv_b1.reshape(1, -1))

    chunk_quant = _NW * _CH * 4
    e_pad = ((e + chunk_quant - 1) // chunk_quant) * chunk_quant
    pad = e_pad - e
    row_g = jnp.concatenate([row, jnp.zeros((pad,), jnp.int32)]).reshape(-1, _CH)
    col_g = jnp.concatenate([col, jnp.zeros((pad,), jnp.int32)]).reshape(-1, _CH)
    ea_pad = jnp.pad(edge_attr.T, ((0, 0), (0, pad)))

    g1 = _sc_gather_pair(p, q, col_g, row_g)
    msg = _edge_mlp1(g1, ea_pad, we1, cv_W2.T, cv_b2.reshape(1, -1), 3200)

    n_pad = ((n + 1 + _NS * 8 - 1) // (_NS * 8)) * (_NS * 8)
    col_s = jnp.concatenate(
        [col, jnp.full((pad,), n, jnp.int32)]).reshape(-1, _CH)
    hn_parts = _sc_segsum(msg, col_s, n_pad)

    r, s = _node_stage_e(hn_parts[:n], hn_parts[n_pad:n_pad + n],
                         xn, wr, ws, eg_b1.reshape(1, -1))
    g2 = _sc_gather_pair(r, s, row_g, col_g)
    out = _edge_mlp2(g2, ea_pad, we2, eg_W2, eg_b2.reshape(1, 1), 3200)
    return out.reshape(e_pad)[:e]


# speedup vs baseline: 1.5115x; 1.5115x over previous
import functools

import jax
import jax.numpy as jnp
from jax import lax
from jax.experimental import pallas as pl
from jax.experimental.pallas import tpu as pltpu
from jax.experimental.pallas import tpu_sc as plsc

_EPS = 1e-5

_NC = 2
_NS = 16
_NW = _NC * _NS
_CH = 128


def _sc_gather_pair(a_tbl, b_tbl, ia2d, ib2d, kk=4):
    ncr, ch = ia2d.shape
    dim = a_tbl.shape[1]
    dt = a_tbl.dtype
    e_pad = ncr * ch
    cr_per_w = ncr // _NW
    nblk = cr_per_w // kk
    mesh = plsc.VectorSubcoreMesh(core_axis_name="c", subcore_axis_name="s",
                                  num_cores=_NC, num_subcores=_NS)

    @functools.partial(
        pl.kernel,
        out_type=jax.ShapeDtypeStruct((e_pad, 2 * dim), dt),
        mesh=mesh,
        scratch_types=[
            pltpu.VMEM((kk, ch), jnp.int32),
            pltpu.VMEM((kk, ch), jnp.int32),
            pltpu.VMEM((kk * ch, dim), dt),
            pltpu.VMEM((kk * ch, dim), dt),
            pltpu.SemaphoreType.DMA,
            pltpu.SemaphoreType.DMA,
        ],
        compiler_params=pltpu.CompilerParams(use_tc_tiling_on_sc=False),
    )
    def k(a_hbm, b_hbm, ia_hbm, ib_hbm, out_hbm, ia_v, ib_v, ra_v, rb_v,
          sa, sb):
        wid = lax.axis_index("s") * _NC + lax.axis_index("c")

        def blk(b, carry):
            cr = wid * cr_per_w + b * kk
            pltpu.sync_copy(ia_hbm.at[pl.ds(cr, kk)], ia_v)
            pltpu.sync_copy(ib_hbm.at[pl.ds(cr, kk)], ib_v)
            da = [pltpu.async_copy(a_hbm.at[ia_v.at[j]],
                                   ra_v.at[pl.ds(j * ch, ch)], sa)
                  for j in range(kk)]
            db = [pltpu.async_copy(b_hbm.at[ib_v.at[j]],
                                   rb_v.at[pl.ds(j * ch, ch)], sb)
                  for j in range(kk)]
            for dsc in da:
                dsc.wait()
            for dsc in db:
                dsc.wait()
            pltpu.sync_copy(
                ra_v, out_hbm.at[pl.ds(cr * ch, kk * ch), pl.ds(0, dim)])
            pltpu.sync_copy(
                rb_v, out_hbm.at[pl.ds(cr * ch, kk * ch), pl.ds(dim, dim)])
            return carry

        lax.fori_loop(0, nblk, blk, 0)

    return k(a_tbl, b_tbl, ia2d, ib2d)


def _sc_segsum(msg, col2d, n_pad, kk=4):
    ncr, ch = col2d.shape
    dim = msg.shape[1]
    stripe = n_pad // _NS
    cr_per_core = ncr // _NC
    cr_per_w = cr_per_core // _NS
    nblk = cr_per_w // kk
    zeros = jnp.zeros((stripe, dim), jnp.float32)
    mesh = plsc.VectorSubcoreMesh(core_axis_name="c", subcore_axis_name="s",
                                  num_cores=_NC, num_subcores=_NS)

    @functools.partial(
        pl.kernel,
        out_type=jax.ShapeDtypeStruct((_NC * n_pad, dim), jnp.float32),
        mesh=mesh,
        scratch_types=[
            pltpu.VMEM((kk, ch), jnp.int32),
            pltpu.VMEM((kk * ch, dim), jnp.float32),
            pltpu.VMEM_SHARED((n_pad, dim), jnp.float32),
        ],
        compiler_params=pltpu.CompilerParams(use_tc_tiling_on_sc=False),
    )
    def k(msg_hbm, col_hbm, z_hbm, out_hbm, ci_v, mb_v, acc_sh):
        cid = lax.axis_index("c")
        sid = lax.axis_index("s")
        pltpu.sync_copy(z_hbm, acc_sh.at[pl.ds(sid * stripe, stripe)])
        plsc.subcore_barrier()

        def blk(b, carry):
            cr = cid * cr_per_core + sid * cr_per_w + b * kk
            pltpu.sync_copy(col_hbm.at[pl.ds(cr, kk)], ci_v)
            pltpu.sync_copy(msg_hbm.at[pl.ds(cr * ch, kk * ch)], mb_v)
            for j in range(kk):
                pltpu.sync_copy(mb_v.at[pl.ds(j * ch, ch)],
                                acc_sh.at[ci_v.at[j]], add=True)
            return carry

        lax.fori_loop(0, nblk, blk, 0)
        plsc.subcore_barrier()
        pltpu.sync_copy(acc_sh.at[pl.ds(sid * stripe, stripe)],
                        out_hbm.at[pl.ds(cid * n_pad + sid * stripe, stripe)])

    return k(msg, col2d, zeros)


_BN = 5000


def _stats_kernel(x_ref, sx_ref, sxx_ref):
    x = x_ref[...]

    @pl.when(pl.program_id(0) == 0)
    def _():
        sx_ref[...] = jnp.zeros_like(sx_ref)
        sxx_ref[...] = jnp.zeros_like(sxx_ref)

    sx_ref[...] += jnp.sum(x, axis=0, keepdims=True)
    sxx_ref[...] += jnp.sum(x * x, axis=0, keepdims=True)


def _node_stats(x):
    n, d = x.shape
    return pl.pallas_call(
        _stats_kernel,
        grid=(n // _BN,),
        in_specs=[pl.BlockSpec((_BN, d), lambda i: (i, 0))],
        out_specs=[pl.BlockSpec((1, d), lambda i: (0, 0)),
                   pl.BlockSpec((1, d), lambda i: (0, 0))],
        out_shape=[jax.ShapeDtypeStruct((1, d), jnp.float32),
                   jax.ShapeDtypeStruct((1, d), jnp.float32)],
    )(x)


def _node_stage_a_kernel(x_ref, sx_ref, sxx_ref, gamma_ref, beta_ref,
                         w1_ref, b1_ref, w2_ref, b2_ref, wa_ref, wb_ref,
                         cb1_ref, n_scale_ref, xout_ref, p_ref, q_ref):
    inv_n = n_scale_ref[0, 0]
    mean = sx_ref[...] * inv_n
    var = sxx_ref[...] * inv_n - mean * mean
    x = x_ref[...]
    xn = (x - mean) * lax.rsqrt(var + _EPS) * gamma_ref[...] + beta_ref[...]
    h = jnp.maximum(jnp.dot(xn, w1_ref[...], preferred_element_type=jnp.float32)
                    + b1_ref[...], 0.0)
    h = jnp.tanh(jnp.dot(h, w2_ref[...], preferred_element_type=jnp.float32)
                 + b2_ref[...])
    feat = jnp.concatenate([h, xn], axis=-1)
    xout_ref[...] = xn
    p_ref[...] = (jnp.dot(feat, wa_ref[...], preferred_element_type=jnp.float32)
                  + cb1_ref[...])
    q_ref[...] = jnp.dot(feat, wb_ref[...], preferred_element_type=jnp.float32)


def _node_stage_a(x, gamma, beta, w1t, b1, w2t, b2, wa, wb, cb1):
    n, d = x.shape
    hd = w1t.shape[1]
    sx, sxx = _node_stats(x)
    n_scale = jnp.full((1, 1), 1.0 / n, dtype=jnp.float32)
    small = lambda s: pl.BlockSpec(s, lambda i: (0,) * len(s))
    return pl.pallas_call(
        _node_stage_a_kernel,
        grid=(n // _BN,),
        in_specs=[
            pl.BlockSpec((_BN, d), lambda i: (i, 0)),
            small((1, d)), small((1, d)), small((1, d)), small((1, d)),
            small(w1t.shape), small(b1.shape), small(w2t.shape),
            small(b2.shape), small(wa.shape), small(wb.shape),
            small(cb1.shape), small((1, 1)),
        ],
        out_specs=[pl.BlockSpec((_BN, d), lambda i: (i, 0)),
                   pl.BlockSpec((_BN, 2 * hd), lambda i: (i, 0)),
                   pl.BlockSpec((_BN, 2 * hd), lambda i: (i, 0))],
        out_shape=[
            jax.ShapeDtypeStruct((n, d), jnp.float32),
            jax.ShapeDtypeStruct((n, 2 * hd), jnp.float32),
            jax.ShapeDtypeStruct((n, 2 * hd), jnp.float32),
        ],
    )(x, sx, sxx, gamma, beta, w1t, b1, w2t, b2, wa, wb, cb1, n_scale)


def _edge_mlp1_kernel(g_ref, ea_ref, we_ref, w2_ref, b2_ref, out_ref):
    g = g_ref[...].astype(jnp.float32)
    hdim = g.shape[1] // 2
    eac = lax.dot_general(ea_ref[...], we_ref[...], (((0,), (0,)), ((), ())),
                          preferred_element_type=jnp.float32)
    h = jnp.maximum(g[:, :hdim] + g[:, hdim:] + eac, 0.0)
    out_ref[...] = jnp.tanh(
        jnp.dot(h, w2_ref[...], preferred_element_type=jnp.float32) + b2_ref[...])


def _edge_mlp1(g, ea, we, w2t, b2, block_e):
    e, gdim = g.shape
    odim = w2t.shape[1]
    grid = e // block_e
    return pl.pallas_call(
        _edge_mlp1_kernel,
        grid=(grid,),
        in_specs=[
            pl.BlockSpec((block_e, gdim), lambda i: (i, 0)),
            pl.BlockSpec((ea.shape[0], block_e), lambda i: (0, i)),
            pl.BlockSpec(we.shape, lambda i: (0, 0)),
            pl.BlockSpec(w2t.shape, lambda i: (0, 0)),
            pl.BlockSpec(b2.shape, lambda i: (0, 0)),
        ],
        out_specs=pl.BlockSpec((block_e, odim), lambda i: (i, 0)),
        out_shape=jax.ShapeDtypeStruct((e, odim), jnp.float32),
    )(g, ea, we, w2t, b2)


def _edge_mlp2_kernel(g_ref, ea_ref, we_ref, w2_ref, b2_ref, out_ref):
    g = g_ref[...].astype(jnp.float32)
    hdim = g.shape[1] // 2
    eac = lax.dot_general(ea_ref[...], we_ref[...], (((0,), (0,)), ((), ())),
                          preferred_element_type=jnp.float32)
    h = jnp.maximum(g[:, :hdim] + g[:, hdim:] + eac, 0.0)
    z = lax.dot_general(w2_ref[...], h, (((1,), (1,)), ((), ())),
                        preferred_element_type=jnp.float32) + b2_ref[...]
    out_ref[...] = jax.nn.sigmoid(z)


def _edge_mlp2(g, ea, we, w2t, b2, block_e):
    e, gdim = g.shape
    grid = e // block_e
    return pl.pallas_call(
        _edge_mlp2_kernel,
        grid=(grid,),
        in_specs=[
            pl.BlockSpec((block_e, gdim), lambda i: (i, 0)),
            pl.BlockSpec((ea.shape[0], block_e), lambda i: (0, i)),
            pl.BlockSpec(we.shape, lambda i: (0, 0)),
            pl.BlockSpec(w2t.shape, lambda i: (0, 0)),
            pl.BlockSpec(b2.shape, lambda i: (0, 0)),
        ],
        out_specs=pl.BlockSpec((1, block_e), lambda i: (0, i)),
        out_shape=jax.ShapeDtypeStruct((1, e), jnp.float32),
    )(g, ea, we, w2t, b2)


def _node_stage_e_kernel(hn0_ref, hn1_ref, x_ref, wr_ref, ws_ref, eb1_ref,
                         r_ref, s_ref):
    feat2 = jnp.concatenate([hn0_ref[...] + hn1_ref[...], x_ref[...]], axis=-1)
    r_ref[...] = jnp.dot(feat2, wr_ref[...], preferred_element_type=jnp.float32)
    s_ref[...] = (jnp.dot(feat2, ws_ref[...], preferred_element_type=jnp.float32)
                  + eb1_ref[...])


def _node_stage_e(hn0, hn1, xn, wr, ws, eb1):
    n = xn.shape[0]
    hdim = wr.shape[1]
    small = lambda s: pl.BlockSpec(s, lambda i: (0,) * len(s))
    return pl.pallas_call(
        _node_stage_e_kernel,
        grid=(n // _BN,),
        in_specs=[
            pl.BlockSpec((_BN, hn0.shape[1]), lambda i: (i, 0)),
            pl.BlockSpec((_BN, hn1.shape[1]), lambda i: (i, 0)),
            pl.BlockSpec((_BN, xn.shape[1]), lambda i: (i, 0)),
            small(wr.shape), small(ws.shape), small(eb1.shape),
        ],
        out_specs=[pl.BlockSpec((_BN, hdim), lambda i: (i, 0)),
                   pl.BlockSpec((_BN, hdim), lambda i: (i, 0))],
        out_shape=[
            jax.ShapeDtypeStruct((n, hdim), jnp.float32),
            jax.ShapeDtypeStruct((n, hdim), jnp.float32),
        ],
    )(hn0, hn1, xn, wr, ws, eb1)


def kernel(x, edge_index, edge_attr, bn_gamma, bn_beta,
           in_W1, in_b1, in_W2, in_b2,
           cv_W1, cv_b1, cv_W2, cv_b2,
           eg_W1, eg_b1, eg_W2, eg_b2):
    n, d = x.shape
    e = edge_index.shape[1]
    hd = in_W1.shape[0]
    fd = hd + d

    row = edge_index[0]
    col = edge_index[1]

    wa1 = (cv_W1[:, :fd] - cv_W1[:, fd:2 * fd]).T
    wb1 = cv_W1[:, fd:2 * fd].T
    we1 = cv_W1[:, 2 * fd:].T
    wr = eg_W1[:, :fd].T
    ws = eg_W1[:, fd:2 * fd].T
    we2 = eg_W1[:, 2 * fd:].T

    xn, p, q = _node_stage_a(
        x, bn_gamma.reshape(1, d), bn_beta.reshape(1, d),
        in_W1.T, in_b1.reshape(1, hd), in_W2.T, in_b2.reshape(1, hd),
        wa1, wb1, cv_b1.reshape(1, -1))

    chunk_quant = _NW * _CH * 4
    e_pad = ((e + chunk_quant - 1) // chunk_quant) * chunk_quant
    pad = e_pad - e
    row_g = jnp.concatenate([row, jnp.zeros((pad,), jnp.int32)]).reshape(-1, _CH)
    col_g = jnp.concatenate([col, jnp.zeros((pad,), jnp.int32)]).reshape(-1, _CH)
    ea_pad = jnp.pad(edge_attr.T, ((0, 0), (0, pad)))

    g1 = _sc_gather_pair(p, q, col_g, row_g)
    msg = _edge_mlp1(g1, ea_pad, we1, cv_W2.T, cv_b2.reshape(1, -1), 3200)

    n_pad = ((n + 1 + _NS * 8 - 1) // (_NS * 8)) * (_NS * 8)
    col_s = jnp.concatenate(
        [col, jnp.full((pad,), n, jnp.int32)]).reshape(-1, _CH)
    hn_parts = _sc_segsum(msg, col_s, n_pad)

    r, s = _node_stage_e(hn_parts[:n], hn_parts[n_pad:n_pad + n],
                         xn, wr, ws, eg_b1.reshape(1, -1))
    g2 = _sc_gather_pair(r, s, row_g, col_g)
    out = _edge_mlp2(g2, ea_pad, we2, eg_W2, eg_b2.reshape(1, 1), 3200)
    return out.reshape(e_pad)[:e]


# speedup vs baseline: 1.5601x; 1.0321x over previous
import functools

import jax
import jax.numpy as jnp
from jax import lax
from jax.experimental import pallas as pl
from jax.experimental.pallas import tpu as pltpu
from jax.experimental.pallas import tpu_sc as plsc

_EPS = 1e-5

_NC = 2
_NS = 16
_NW = _NC * _NS
_CH = 128


def _sc_gather_pair(a_tbl, b_tbl, ia2d, ib2d, kk=7):
    ncr, ch = ia2d.shape
    dim = a_tbl.shape[1]
    dt = a_tbl.dtype
    e_pad = ncr * ch
    cr_per_w = ncr // _NW
    nblk = cr_per_w // kk
    mesh = plsc.VectorSubcoreMesh(core_axis_name="c", subcore_axis_name="s",
                                  num_cores=_NC, num_subcores=_NS)

    @functools.partial(
        pl.kernel,
        out_type=jax.ShapeDtypeStruct((e_pad, 2 * dim), dt),
        mesh=mesh,
        scratch_types=[
            pltpu.VMEM((kk, ch), jnp.int32),
            pltpu.VMEM((kk, ch), jnp.int32),
            pltpu.VMEM((kk * ch, dim), dt),
            pltpu.VMEM((kk * ch, dim), dt),
            pltpu.SemaphoreType.DMA,
            pltpu.SemaphoreType.DMA,
        ],
        compiler_params=pltpu.CompilerParams(use_tc_tiling_on_sc=False),
    )
    def k(a_hbm, b_hbm, ia_hbm, ib_hbm, out_hbm, ia_v, ib_v, ra_v, rb_v,
          sa, sb):
        wid = lax.axis_index("s") * _NC + lax.axis_index("c")

        def blk(b, carry):
            cr = wid * cr_per_w + b * kk
            pltpu.sync_copy(ia_hbm.at[pl.ds(cr, kk)], ia_v)
            pltpu.sync_copy(ib_hbm.at[pl.ds(cr, kk)], ib_v)
            da = [pltpu.async_copy(a_hbm.at[ia_v.at[j]],
                                   ra_v.at[pl.ds(j * ch, ch)], sa)
                  for j in range(kk)]
            db = [pltpu.async_copy(b_hbm.at[ib_v.at[j]],
                                   rb_v.at[pl.ds(j * ch, ch)], sb)
                  for j in range(kk)]
            for dsc in da:
                dsc.wait()
            for dsc in db:
                dsc.wait()
            pltpu.sync_copy(
                ra_v, out_hbm.at[pl.ds(cr * ch, kk * ch), pl.ds(0, dim)])
            pltpu.sync_copy(
                rb_v, out_hbm.at[pl.ds(cr * ch, kk * ch), pl.ds(dim, dim)])
            return carry

        lax.fori_loop(0, nblk, blk, 0)

    return k(a_tbl, b_tbl, ia2d, ib2d)


def _sc_segsum(msg, col2d, n_pad, kk=4):
    ncr, ch = col2d.shape
    dim = msg.shape[1]
    stripe = n_pad // _NS
    cr_per_core = ncr // _NC
    cr_per_w = cr_per_core // _NS
    nblk = cr_per_w // kk
    zeros = jnp.zeros((stripe, dim), jnp.float32)
    mesh = plsc.VectorSubcoreMesh(core_axis_name="c", subcore_axis_name="s",
                                  num_cores=_NC, num_subcores=_NS)

    @functools.partial(
        pl.kernel,
        out_type=jax.ShapeDtypeStruct((_NC * n_pad, dim), jnp.float32),
        mesh=mesh,
        scratch_types=[
            pltpu.VMEM((kk, ch), jnp.int32),
            pltpu.VMEM((kk * ch, dim), jnp.float32),
            pltpu.VMEM_SHARED((n_pad, dim), jnp.float32),
        ],
        compiler_params=pltpu.CompilerParams(use_tc_tiling_on_sc=False),
    )
    def k(msg_hbm, col_hbm, z_hbm, out_hbm, ci_v, mb_v, acc_sh):
        cid = lax.axis_index("c")
        sid = lax.axis_index("s")
        pltpu.sync_copy(z_hbm, acc_sh.at[pl.ds(sid * stripe, stripe)])
        plsc.subcore_barrier()

        def blk(b, carry):
            cr = cid * cr_per_core + sid * cr_per_w + b * kk
            pltpu.sync_copy(col_hbm.at[pl.ds(cr, kk)], ci_v)
            pltpu.sync_copy(msg_hbm.at[pl.ds(cr * ch, kk * ch)], mb_v)
            for j in range(kk):
                pltpu.sync_copy(mb_v.at[pl.ds(j * ch, ch)],
                                acc_sh.at[ci_v.at[j]], add=True)
            return carry

        lax.fori_loop(0, nblk, blk, 0)
        plsc.subcore_barrier()
        pltpu.sync_copy(acc_sh.at[pl.ds(sid * stripe, stripe)],
                        out_hbm.at[pl.ds(cid * n_pad + sid * stripe, stripe)])

    return k(msg, col2d, zeros)


_BN = 5000


def _stats_kernel(x_ref, sx_ref, sxx_ref):
    x = x_ref[...]

    @pl.when(pl.program_id(0) == 0)
    def _():
        sx_ref[...] = jnp.zeros_like(sx_ref)
        sxx_ref[...] = jnp.zeros_like(sxx_ref)

    sx_ref[...] += jnp.sum(x, axis=0, keepdims=True)
    sxx_ref[...] += jnp.sum(x * x, axis=0, keepdims=True)


def _node_stats(x):
    n, d = x.shape
    return pl.pallas_call(
        _stats_kernel,
        grid=(n // _BN,),
        in_specs=[pl.BlockSpec((_BN, d), lambda i: (i, 0))],
        out_specs=[pl.BlockSpec((1, d), lambda i: (0, 0)),
                   pl.BlockSpec((1, d), lambda i: (0, 0))],
        out_shape=[jax.ShapeDtypeStruct((1, d), jnp.float32),
                   jax.ShapeDtypeStruct((1, d), jnp.float32)],
    )(x)


def _node_stage_a_kernel(x_ref, sx_ref, sxx_ref, gamma_ref, beta_ref,
                         w1_ref, b1_ref, w2_ref, b2_ref, wa_ref, wb_ref,
                         cb1_ref, n_scale_ref, xout_ref, p_ref, q_ref):
    inv_n = n_scale_ref[0, 0]
    mean = sx_ref[...] * inv_n
    var = sxx_ref[...] * inv_n - mean * mean
    x = x_ref[...]
    xn = (x - mean) * lax.rsqrt(var + _EPS) * gamma_ref[...] + beta_ref[...]
    h = jnp.maximum(jnp.dot(xn, w1_ref[...], preferred_element_type=jnp.float32)
                    + b1_ref[...], 0.0)
    h = jnp.tanh(jnp.dot(h, w2_ref[...], preferred_element_type=jnp.float32)
                 + b2_ref[...])
    feat = jnp.concatenate([h, xn], axis=-1)
    xout_ref[...] = xn
    p_ref[...] = (jnp.dot(feat, wa_ref[...], preferred_element_type=jnp.float32)
                  + cb1_ref[...])
    q_ref[...] = jnp.dot(feat, wb_ref[...], preferred_element_type=jnp.float32)


def _node_stage_a(x, gamma, beta, w1t, b1, w2t, b2, wa, wb, cb1):
    n, d = x.shape
    hd = w1t.shape[1]
    sx, sxx = _node_stats(x)
    n_scale = jnp.full((1, 1), 1.0 / n, dtype=jnp.float32)
    small = lambda s: pl.BlockSpec(s, lambda i: (0,) * len(s))
    return pl.pallas_call(
        _node_stage_a_kernel,
        grid=(n // _BN,),
        in_specs=[
            pl.BlockSpec((_BN, d), lambda i: (i, 0)),
            small((1, d)), small((1, d)), small((1, d)), small((1, d)),
            small(w1t.shape), small(b1.shape), small(w2t.shape),
            small(b2.shape), small(wa.shape), small(wb.shape),
            small(cb1.shape), small((1, 1)),
        ],
        out_specs=[pl.BlockSpec((_BN, d), lambda i: (i, 0)),
                   pl.BlockSpec((_BN, 2 * hd), lambda i: (i, 0)),
                   pl.BlockSpec((_BN, 2 * hd), lambda i: (i, 0))],
        out_shape=[
            jax.ShapeDtypeStruct((n, d), jnp.float32),
            jax.ShapeDtypeStruct((n, 2 * hd), jnp.float32),
            jax.ShapeDtypeStruct((n, 2 * hd), jnp.float32),
        ],
    )(x, sx, sxx, gamma, beta, w1t, b1, w2t, b2, wa, wb, cb1, n_scale)


def _edge_mlp1_kernel(g_ref, ea_ref, we_ref, w2_ref, b2_ref, out_ref):
    g = g_ref[...].astype(jnp.float32)
    hdim = g.shape[1] // 2
    eac = lax.dot_general(ea_ref[...], we_ref[...], (((0,), (0,)), ((), ())),
                          preferred_element_type=jnp.float32)
    h = jnp.maximum(g[:, :hdim] + g[:, hdim:] + eac, 0.0)
    out_ref[...] = jnp.tanh(
        jnp.dot(h, w2_ref[...], preferred_element_type=jnp.float32) + b2_ref[...])


def _edge_mlp1(g, ea, we, w2t, b2, block_e):
    e, gdim = g.shape
    odim = w2t.shape[1]
    grid = e // block_e
    return pl.pallas_call(
        _edge_mlp1_kernel,
        grid=(grid,),
        in_specs=[
            pl.BlockSpec((block_e, gdim), lambda i: (i, 0)),
            pl.BlockSpec((ea.shape[0], block_e), lambda i: (0, i)),
            pl.BlockSpec(we.shape, lambda i: (0, 0)),
            pl.BlockSpec(w2t.shape, lambda i: (0, 0)),
            pl.BlockSpec(b2.shape, lambda i: (0, 0)),
        ],
        out_specs=pl.BlockSpec((block_e, odim), lambda i: (i, 0)),
        out_shape=jax.ShapeDtypeStruct((e, odim), jnp.float32),
    )(g, ea, we, w2t, b2)


def _edge_mlp2_kernel(g_ref, ea_ref, we_ref, w2_ref, b2_ref, out_ref):
    g = g_ref[...].astype(jnp.float32)
    hdim = g.shape[1] // 2
    eac = lax.dot_general(ea_ref[...], we_ref[...], (((0,), (0,)), ((), ())),
                          preferred_element_type=jnp.float32)
    h = jnp.maximum(g[:, :hdim] + g[:, hdim:] + eac, 0.0)
    z = lax.dot_general(w2_ref[...], h, (((1,), (1,)), ((), ())),
                        preferred_element_type=jnp.float32) + b2_ref[...]
    out_ref[...] = jax.nn.sigmoid(z)


def _edge_mlp2(g, ea, we, w2t, b2, block_e):
    e, gdim = g.shape
    grid = e // block_e
    return pl.pallas_call(
        _edge_mlp2_kernel,
        grid=(grid,),
        in_specs=[
            pl.BlockSpec((block_e, gdim), lambda i: (i, 0)),
            pl.BlockSpec((ea.shape[0], block_e), lambda i: (0, i)),
            pl.BlockSpec(we.shape, lambda i: (0, 0)),
            pl.BlockSpec(w2t.shape, lambda i: (0, 0)),
            pl.BlockSpec(b2.shape, lambda i: (0, 0)),
        ],
        out_specs=pl.BlockSpec((1, block_e), lambda i: (0, i)),
        out_shape=jax.ShapeDtypeStruct((1, e), jnp.float32),
    )(g, ea, we, w2t, b2)


def _node_stage_e_kernel(hn0_ref, hn1_ref, x_ref, wr_ref, ws_ref, eb1_ref,
                         r_ref, s_ref):
    feat2 = jnp.concatenate([hn0_ref[...] + hn1_ref[...], x_ref[...]], axis=-1)
    r_ref[...] = jnp.dot(feat2, wr_ref[...], preferred_element_type=jnp.float32)
    s_ref[...] = (jnp.dot(feat2, ws_ref[...], preferred_element_type=jnp.float32)
                  + eb1_ref[...])


def _node_stage_e(hn0, hn1, xn, wr, ws, eb1):
    n = xn.shape[0]
    hdim = wr.shape[1]
    small = lambda s: pl.BlockSpec(s, lambda i: (0,) * len(s))
    return pl.pallas_call(
        _node_stage_e_kernel,
        grid=(n // _BN,),
        in_specs=[
            pl.BlockSpec((_BN, hn0.shape[1]), lambda i: (i, 0)),
            pl.BlockSpec((_BN, hn1.shape[1]), lambda i: (i, 0)),
            pl.BlockSpec((_BN, xn.shape[1]), lambda i: (i, 0)),
            small(wr.shape), small(ws.shape), small(eb1.shape),
        ],
        out_specs=[pl.BlockSpec((_BN, hdim), lambda i: (i, 0)),
                   pl.BlockSpec((_BN, hdim), lambda i: (i, 0))],
        out_shape=[
            jax.ShapeDtypeStruct((n, hdim), jnp.float32),
            jax.ShapeDtypeStruct((n, hdim), jnp.float32),
        ],
    )(hn0, hn1, xn, wr, ws, eb1)


def kernel(x, edge_index, edge_attr, bn_gamma, bn_beta,
           in_W1, in_b1, in_W2, in_b2,
           cv_W1, cv_b1, cv_W2, cv_b2,
           eg_W1, eg_b1, eg_W2, eg_b2):
    n, d = x.shape
    e = edge_index.shape[1]
    hd = in_W1.shape[0]
    fd = hd + d

    row = edge_index[0]
    col = edge_index[1]

    wa1 = (cv_W1[:, :fd] - cv_W1[:, fd:2 * fd]).T
    wb1 = cv_W1[:, fd:2 * fd].T
    we1 = cv_W1[:, 2 * fd:].T
    wr = eg_W1[:, :fd].T
    ws = eg_W1[:, fd:2 * fd].T
    we2 = eg_W1[:, 2 * fd:].T

    xn, p, q = _node_stage_a(
        x, bn_gamma.reshape(1, d), bn_beta.reshape(1, d),
        in_W1.T, in_b1.reshape(1, hd), in_W2.T, in_b2.reshape(1, hd),
        wa1, wb1, cv_b1.reshape(1, -1))

    chunk_quant = _NW * _CH * 4
    e_pad = ((e + chunk_quant - 1) // chunk_quant) * chunk_quant
    pad = e_pad - e
    row_g = jnp.concatenate([row, jnp.zeros((pad,), jnp.int32)]).reshape(-1, _CH)
    col_g = jnp.concatenate([col, jnp.zeros((pad,), jnp.int32)]).reshape(-1, _CH)
    ea_pad = jnp.pad(edge_attr.T, ((0, 0), (0, pad)))

    g1 = _sc_gather_pair(p, q, col_g, row_g)
    msg = _edge_mlp1(g1, ea_pad, we1, cv_W2.T, cv_b2.reshape(1, -1), 3200)

    n_pad = ((n + 1 + _NS * 8 - 1) // (_NS * 8)) * (_NS * 8)
    col_s = jnp.concatenate(
        [col, jnp.full((pad,), n, jnp.int32)]).reshape(-1, _CH)
    hn_parts = _sc_segsum(msg, col_s, n_pad)

    r, s = _node_stage_e(hn_parts[:n], hn_parts[n_pad:n_pad + n],
                         xn, wr, ws, eg_b1.reshape(1, -1))
    g2 = _sc_gather_pair(r, s, row_g, col_g)
    out = _edge_mlp2(g2, ea_pad, we2, eg_W2, eg_b2.reshape(1, 1), 3200)
    return out.reshape(e_pad)[:e]


# speedup vs baseline: 1.7324x; 1.1105x over previous
import functools

import jax
import jax.numpy as jnp
from jax import lax
from jax.experimental import pallas as pl
from jax.experimental.pallas import tpu as pltpu
from jax.experimental.pallas import tpu_sc as plsc

_EPS = 1e-5

_NC = 2
_NS = 16
_NW = _NC * _NS
_CH = 128


def _sc_gather_pair(a_tbl, b_tbl, ia2d, ib2d, kk=7):
    ncr, ch = ia2d.shape
    dim = a_tbl.shape[1]
    dt = a_tbl.dtype
    e_pad = ncr * ch
    cr_per_w = ncr // _NW
    nblk = cr_per_w // kk
    mesh = plsc.VectorSubcoreMesh(core_axis_name="c", subcore_axis_name="s",
                                  num_cores=_NC, num_subcores=_NS)

    @functools.partial(
        pl.kernel,
        out_type=jax.ShapeDtypeStruct((e_pad, 2 * dim), dt),
        mesh=mesh,
        scratch_types=[
            pltpu.VMEM((kk, ch), jnp.int32),
            pltpu.VMEM((kk, ch), jnp.int32),
            pltpu.VMEM((kk * ch, dim), dt),
            pltpu.VMEM((kk * ch, dim), dt),
            pltpu.SemaphoreType.DMA,
            pltpu.SemaphoreType.DMA,
        ],
        compiler_params=pltpu.CompilerParams(use_tc_tiling_on_sc=False),
    )
    def k(a_hbm, b_hbm, ia_hbm, ib_hbm, out_hbm, ia_v, ib_v, ra_v, rb_v,
          sa, sb):
        wid = lax.axis_index("s") * _NC + lax.axis_index("c")

        def blk(b, carry):
            cr = wid * cr_per_w + b * kk
            pltpu.sync_copy(ia_hbm.at[pl.ds(cr, kk)], ia_v)
            pltpu.sync_copy(ib_hbm.at[pl.ds(cr, kk)], ib_v)
            da = [pltpu.async_copy(a_hbm.at[ia_v.at[j]],
                                   ra_v.at[pl.ds(j * ch, ch)], sa)
                  for j in range(kk)]
            db = [pltpu.async_copy(b_hbm.at[ib_v.at[j]],
                                   rb_v.at[pl.ds(j * ch, ch)], sb)
                  for j in range(kk)]
            for dsc in da:
                dsc.wait()
            for dsc in db:
                dsc.wait()
            pltpu.sync_copy(
                ra_v, out_hbm.at[pl.ds(cr * ch, kk * ch), pl.ds(0, dim)])
            pltpu.sync_copy(
                rb_v, out_hbm.at[pl.ds(cr * ch, kk * ch), pl.ds(dim, dim)])
            return carry

        lax.fori_loop(0, nblk, blk, 0)

    return k(a_tbl, b_tbl, ia2d, ib2d)


def _sc_segsum(msg_packed, col2d, n_pad, dim, kk=4):
    ncr, ch = col2d.shape
    gpr = 128 // dim
    rpc = ch // gpr
    stripe = n_pad // _NS
    cr_per_core = ncr // _NC
    cr_per_w = cr_per_core // _NS
    nblk = cr_per_w // kk
    zeros = jnp.zeros((stripe, dim), jnp.float32)
    mesh = plsc.VectorSubcoreMesh(core_axis_name="c", subcore_axis_name="s",
                                  num_cores=_NC, num_subcores=_NS)

    @functools.partial(
        pl.kernel,
        out_type=jax.ShapeDtypeStruct((_NC * n_pad, dim), jnp.float32),
        mesh=mesh,
        scratch_types=[
            pltpu.VMEM((kk, ch), jnp.int32),
            pltpu.VMEM((kk * ch, dim), jnp.float32),
            pltpu.VMEM_SHARED((n_pad, dim), jnp.float32),
        ],
        compiler_params=pltpu.CompilerParams(use_tc_tiling_on_sc=False),
    )
    def k(msg_hbm, col_hbm, z_hbm, out_hbm, ci_v, grp_v, acc_sh):
        cid = lax.axis_index("c")
        sid = lax.axis_index("s")
        pltpu.sync_copy(z_hbm, acc_sh.at[pl.ds(sid * stripe, stripe)])
        plsc.subcore_barrier()

        def blk(b, carry):
            cr = cid * cr_per_core + sid * cr_per_w + b * kk
            mblk = cr // (4 * gpr)
            grp = (cr % (4 * gpr)) // gpr
            pltpu.sync_copy(col_hbm.at[pl.ds(cr, kk)], ci_v)
            pltpu.sync_copy(
                msg_hbm.at[pl.ds(mblk * (kk * ch), kk * ch),
                           pl.ds(grp * dim, dim)],
                grp_v)
            for j in range(kk):
                pltpu.sync_copy(grp_v.at[pl.ds(j * ch, ch)],
                                acc_sh.at[ci_v.at[j]], add=True)
            return carry

        lax.fori_loop(0, nblk, blk, 0)
        plsc.subcore_barrier()
        pltpu.sync_copy(acc_sh.at[pl.ds(sid * stripe, stripe)],
                        out_hbm.at[pl.ds(cid * n_pad + sid * stripe, stripe)])

    return k(msg_packed, col2d, zeros)


_BN = 5000


def _stats_kernel(x_ref, sx_ref, sxx_ref):
    x = x_ref[...]

    @pl.when(pl.program_id(0) == 0)
    def _():
        sx_ref[...] = jnp.zeros_like(sx_ref)
        sxx_ref[...] = jnp.zeros_like(sxx_ref)

    sx_ref[...] += jnp.sum(x, axis=0, keepdims=True)
    sxx_ref[...] += jnp.sum(x * x, axis=0, keepdims=True)


def _node_stats(x):
    n, d = x.shape
    return pl.pallas_call(
        _stats_kernel,
        grid=(n // _BN,),
        in_specs=[pl.BlockSpec((_BN, d), lambda i: (i, 0))],
        out_specs=[pl.BlockSpec((1, d), lambda i: (0, 0)),
                   pl.BlockSpec((1, d), lambda i: (0, 0))],
        out_shape=[jax.ShapeDtypeStruct((1, d), jnp.float32),
                   jax.ShapeDtypeStruct((1, d), jnp.float32)],
    )(x)


def _node_stage_a_kernel(x_ref, sx_ref, sxx_ref, gamma_ref, beta_ref,
                         w1_ref, b1_ref, w2_ref, b2_ref, wa_ref, wb_ref,
                         cb1_ref, n_scale_ref, xout_ref, p_ref, q_ref):
    inv_n = n_scale_ref[0, 0]
    mean = sx_ref[...] * inv_n
    var = sxx_ref[...] * inv_n - mean * mean
    x = x_ref[...]
    xn = (x - mean) * lax.rsqrt(var + _EPS) * gamma_ref[...] + beta_ref[...]
    h = jnp.maximum(jnp.dot(xn, w1_ref[...], preferred_element_type=jnp.float32)
                    + b1_ref[...], 0.0)
    h = jnp.tanh(jnp.dot(h, w2_ref[...], preferred_element_type=jnp.float32)
                 + b2_ref[...])
    feat = jnp.concatenate([h, xn], axis=-1)
    xout_ref[...] = xn
    p_ref[...] = (jnp.dot(feat, wa_ref[...], preferred_element_type=jnp.float32)
                  + cb1_ref[...])
    q_ref[...] = jnp.dot(feat, wb_ref[...], preferred_element_type=jnp.float32)


def _node_stage_a(x, gamma, beta, w1t, b1, w2t, b2, wa, wb, cb1):
    n, d = x.shape
    hd = w1t.shape[1]
    sx, sxx = _node_stats(x)
    n_scale = jnp.full((1, 1), 1.0 / n, dtype=jnp.float32)
    small = lambda s: pl.BlockSpec(s, lambda i: (0,) * len(s))
    return pl.pallas_call(
        _node_stage_a_kernel,
        grid=(n // _BN,),
        in_specs=[
            pl.BlockSpec((_BN, d), lambda i: (i, 0)),
            small((1, d)), small((1, d)), small((1, d)), small((1, d)),
            small(w1t.shape), small(b1.shape), small(w2t.shape),
            small(b2.shape), small(wa.shape), small(wb.shape),
            small(cb1.shape), small((1, 1)),
        ],
        out_specs=[pl.BlockSpec((_BN, d), lambda i: (i, 0)),
                   pl.BlockSpec((_BN, 2 * hd), lambda i: (i, 0)),
                   pl.BlockSpec((_BN, 2 * hd), lambda i: (i, 0))],
        out_shape=[
            jax.ShapeDtypeStruct((n, d), jnp.float32),
            jax.ShapeDtypeStruct((n, 2 * hd), jnp.float32),
            jax.ShapeDtypeStruct((n, 2 * hd), jnp.float32),
        ],
    )(x, sx, sxx, gamma, beta, w1t, b1, w2t, b2, wa, wb, cb1, n_scale)


def _edge_mlp1_kernel(g_ref, ea_ref, we_ref, w2_ref, b2_ref, out_ref):
    g = g_ref[...].astype(jnp.float32)
    hdim = g.shape[1] // 2
    eac = lax.dot_general(ea_ref[...], we_ref[...], (((0,), (0,)), ((), ())),
                          preferred_element_type=jnp.float32)
    h = jnp.maximum(g[:, :hdim] + g[:, hdim:] + eac, 0.0)
    msg = jnp.tanh(
        jnp.dot(h, w2_ref[...], preferred_element_type=jnp.float32) + b2_ref[...])
    seg = msg.shape[0] // 4
    out_ref[...] = jnp.concatenate(
        [msg[i * seg:(i + 1) * seg] for i in range(4)], axis=1)


def _edge_mlp1(g, ea, we, w2t, b2, block_e):
    e, gdim = g.shape
    odim = w2t.shape[1]
    grid = e // block_e
    return pl.pallas_call(
        _edge_mlp1_kernel,
        grid=(grid,),
        in_specs=[
            pl.BlockSpec((block_e, gdim), lambda i: (i, 0)),
            pl.BlockSpec((ea.shape[0], block_e), lambda i: (0, i)),
            pl.BlockSpec(we.shape, lambda i: (0, 0)),
            pl.BlockSpec(w2t.shape, lambda i: (0, 0)),
            pl.BlockSpec(b2.shape, lambda i: (0, 0)),
        ],
        out_specs=pl.BlockSpec((block_e * odim // 128, 128), lambda i: (i, 0)),
        out_shape=jax.ShapeDtypeStruct((e * odim // 128, 128), jnp.float32),
    )(g, ea, we, w2t, b2)


def _edge_mlp2_kernel(g_ref, ea_ref, we_ref, w2_ref, b2_ref, out_ref):
    g = g_ref[...].astype(jnp.float32)
    hdim = g.shape[1] // 2
    eac = lax.dot_general(ea_ref[...], we_ref[...], (((0,), (0,)), ((), ())),
                          preferred_element_type=jnp.float32)
    h = jnp.maximum(g[:, :hdim] + g[:, hdim:] + eac, 0.0)
    z = lax.dot_general(w2_ref[...], h, (((1,), (1,)), ((), ())),
                        preferred_element_type=jnp.float32) + b2_ref[...]
    out_ref[...] = jax.nn.sigmoid(z)


def _edge_mlp2(g, ea, we, w2t, b2, block_e):
    e, gdim = g.shape
    grid = e // block_e
    return pl.pallas_call(
        _edge_mlp2_kernel,
        grid=(grid,),
        in_specs=[
            pl.BlockSpec((block_e, gdim), lambda i: (i, 0)),
            pl.BlockSpec((ea.shape[0], block_e), lambda i: (0, i)),
            pl.BlockSpec(we.shape, lambda i: (0, 0)),
            pl.BlockSpec(w2t.shape, lambda i: (0, 0)),
            pl.BlockSpec(b2.shape, lambda i: (0, 0)),
        ],
        out_specs=pl.BlockSpec((1, block_e), lambda i: (0, i)),
        out_shape=jax.ShapeDtypeStruct((1, e), jnp.float32),
    )(g, ea, we, w2t, b2)


def _node_stage_e_kernel(hn0_ref, hn1_ref, x_ref, wr_ref, ws_ref, eb1_ref,
                         r_ref, s_ref):
    feat2 = jnp.concatenate([hn0_ref[...] + hn1_ref[...], x_ref[...]], axis=-1)
    r_ref[...] = jnp.dot(feat2, wr_ref[...], preferred_element_type=jnp.float32)
    s_ref[...] = (jnp.dot(feat2, ws_ref[...], preferred_element_type=jnp.float32)
                  + eb1_ref[...])


def _node_stage_e(hn0, hn1, xn, wr, ws, eb1):
    n = xn.shape[0]
    hdim = wr.shape[1]
    small = lambda s: pl.BlockSpec(s, lambda i: (0,) * len(s))
    return pl.pallas_call(
        _node_stage_e_kernel,
        grid=(n // _BN,),
        in_specs=[
            pl.BlockSpec((_BN, hn0.shape[1]), lambda i: (i, 0)),
            pl.BlockSpec((_BN, hn1.shape[1]), lambda i: (i, 0)),
            pl.BlockSpec((_BN, xn.shape[1]), lambda i: (i, 0)),
            small(wr.shape), small(ws.shape), small(eb1.shape),
        ],
        out_specs=[pl.BlockSpec((_BN, hdim), lambda i: (i, 0)),
                   pl.BlockSpec((_BN, hdim), lambda i: (i, 0))],
        out_shape=[
            jax.ShapeDtypeStruct((n, hdim), jnp.float32),
            jax.ShapeDtypeStruct((n, hdim), jnp.float32),
        ],
    )(hn0, hn1, xn, wr, ws, eb1)


def kernel(x, edge_index, edge_attr, bn_gamma, bn_beta,
           in_W1, in_b1, in_W2, in_b2,
           cv_W1, cv_b1, cv_W2, cv_b2,
           eg_W1, eg_b1, eg_W2, eg_b2):
    n, d = x.shape
    e = edge_index.shape[1]
    hd = in_W1.shape[0]
    fd = hd + d

    row = edge_index[0]
    col = edge_index[1]

    wa1 = (cv_W1[:, :fd] - cv_W1[:, fd:2 * fd]).T
    wb1 = cv_W1[:, fd:2 * fd].T
    we1 = cv_W1[:, 2 * fd:].T
    wr = eg_W1[:, :fd].T
    ws = eg_W1[:, fd:2 * fd].T
    we2 = eg_W1[:, 2 * fd:].T

    xn, p, q = _node_stage_a(
        x, bn_gamma.reshape(1, d), bn_beta.reshape(1, d),
        in_W1.T, in_b1.reshape(1, hd), in_W2.T, in_b2.reshape(1, hd),
        wa1, wb1, cv_b1.reshape(1, -1))

    chunk_quant = _NW * _CH * 4
    e_pad = ((e + chunk_quant - 1) // chunk_quant) * chunk_quant
    pad = e_pad - e
    row_g = jnp.concatenate([row, jnp.zeros((pad,), jnp.int32)]).reshape(-1, _CH)
    col_g = jnp.concatenate([col, jnp.zeros((pad,), jnp.int32)]).reshape(-1, _CH)
    ea_pad = jnp.pad(edge_attr.T, ((0, 0), (0, pad)))

    g1 = _sc_gather_pair(p, q, col_g, row_g)
    msg = _edge_mlp1(g1, ea_pad, we1, cv_W2.T, cv_b2.reshape(1, -1), 2048)

    n_pad = ((n + 1 + _NS * 8 - 1) // (_NS * 8)) * (_NS * 8)
    col_s = jnp.concatenate(
        [col, jnp.full((pad,), n, jnp.int32)]).reshape(-1, _CH)
    hn_parts = _sc_segsum(msg, col_s, n_pad, hd)

    r, s = _node_stage_e(hn_parts[:n], hn_parts[n_pad:n_pad + n],
                         xn, wr, ws, eg_b1.reshape(1, -1))
    g2 = _sc_gather_pair(r, s, row_g, col_g)
    out = _edge_mlp2(g2, ea_pad, we2, eg_W2, eg_b2.reshape(1, 1), 3200)
    return out.reshape(e_pad)[:e]
